# Initial kernel scaffold; baseline (speedup 1.0000x reference)
import jax
import jax.numpy as jnp
from jax.experimental import pallas as pl


def kernel(prob, bx_dimfull, by_dimfull, bw_dimfull, bh_dimfull):
    raise NotImplementedError("write your pallas kernel here")


# speedup vs baseline: 697.3831x; 697.3831x over previous
import functools

import jax
import jax.numpy as jnp
from jax.experimental import pallas as pl

_P_THRESHOLD = 0.1
_OVERLAP_THRESHOLD = 0.3
_N_MAX_OBJECTS = 20
_LANE = 128


def _nms_body(n_real, p_ref, bx_ref, by_ref, bw_ref, bh_ref,
              op_ref, ox_ref, oy_ref, ow_ref, oh_ref):
    p = p_ref[...]
    bx = bx_ref[...]
    by = by_ref[...]
    bw = bw_ref[...]
    bh = bh_ref[...]
    b, n = p.shape

    x1 = bx - 0.5 * bw
    x3 = bx + 0.5 * bw
    y1 = by - 0.5 * bh
    y3 = by + 0.5 * bh
    area = bw * bh

    col = jax.lax.broadcasted_iota(jnp.int32, (b, n), 1)
    slot = jax.lax.broadcasted_iota(jnp.int32, (b, _LANE), 1)

    possible0 = (p > _P_THRESHOLD).astype(jnp.float32)
    excluded0 = (col >= n_real).astype(jnp.float32)
    zacc = jnp.zeros((b, _LANE), dtype=jnp.float32)

    def body(l, carry):
        possible, excluded, ap, ax, ay, aw, ah = carry
        scores = p * possible
        pmax = jnp.max(scores, axis=1, keepdims=True)
        valid = pmax > 0.0
        m = jnp.min(jnp.where(scores == pmax, col, n), axis=1, keepdims=True)
        m2 = jnp.min(jnp.where(excluded > 0.0, n, col), axis=1, keepdims=True)
        chosen = jnp.where(valid, m, m2)
        sel = (col == chosen).astype(jnp.float32)

        def pick(v):
            return jnp.sum(sel * v, axis=1, keepdims=True)

        bxm = pick(bx)
        bym = pick(by)
        bwm = pick(bw)
        bhm = pick(bh)
        val = jnp.where(valid, pmax, 0.0)

        x1m = bxm - 0.5 * bwm
        x3m = bxm + 0.5 * bwm
        y1m = bym - 0.5 * bhm
        y3m = bym + 0.5 * bhm
        aream = bwm * bhm
        inter = (jnp.maximum(jnp.minimum(x3, x3m) - jnp.maximum(x1, x1m), 0.0)
                 * jnp.maximum(jnp.minimum(y3, y3m) - jnp.maximum(y1, y1m), 0.0))
        keep = jnp.where(inter / jnp.minimum(area, aream) > _OVERLAP_THRESHOLD,
                         0.0, 1.0)
        possible = possible * jnp.where(valid, keep, 1.0)
        excluded = jnp.maximum(excluded, sel)

        at = slot == l
        ap = jnp.where(at, val, ap)
        ax = jnp.where(at, bxm, ax)
        ay = jnp.where(at, bym, ay)
        aw = jnp.where(at, bwm, aw)
        ah = jnp.where(at, bhm, ah)
        return possible, excluded, ap, ax, ay, aw, ah

    _, _, ap, ax, ay, aw, ah = jax.lax.fori_loop(
        0, _N_MAX_OBJECTS, body,
        (possible0, excluded0, zacc, zacc, zacc, zacc, zacc))

    op_ref[...] = ap
    ox_ref[...] = ax
    oy_ref[...] = ay
    ow_ref[...] = aw
    oh_ref[...] = ah


@jax.jit
def kernel(prob, bx_dimfull, by_dimfull, bw_dimfull, bh_dimfull):
    b, n, _ = prob.shape
    n_pad = ((n + _LANE - 1) // _LANE) * _LANE

    def prep(v, fill):
        v = v[..., 0]
        return jnp.pad(v, ((0, 0), (0, n_pad - n)), constant_values=fill)

    p = prep(prob, 0.0)
    bx = prep(bx_dimfull, 0.0)
    by = prep(by_dimfull, 0.0)
    bw = prep(bw_dimfull, 1.0)
    bh = prep(bh_dimfull, 1.0)

    out = jax.ShapeDtypeStruct((b, _LANE), jnp.float32)
    ap, ax, ay, aw, ah = pl.pallas_call(
        functools.partial(_nms_body, n),
        out_shape=(out, out, out, out, out),
    )(p, bx, by, bw, bh)

    k = min(_N_MAX_OBJECTS, n)
    return (ap[:, :k, None], ax[:, :k, None], ay[:, :k, None],
            aw[:, :k, None], ah[:, :k, None])
